# baseline (speedup 1.0000x reference)
import functools

import jax
import jax.numpy as jnp
from jax import lax
from jax.experimental import pallas as pl
from jax.experimental.pallas import tpu as pltpu
from jax.experimental.pallas import tpu_sc as plsc

N_TOK = 4096
D = 1024
E = 6
NC = 1000
BT = 512
CH = 512
NPAIR = 2 * N_TOK
BE = 512
HS_PAD = NPAIR + E * BE
NW = 32
PW = NPAIR // NW
TW = N_TOK // NW
NEG = -1e30
D2 = D // 2

def _mesh():
    return plsc.VectorSubcoreMesh(
        core_axis_name="c", subcore_axis_name="s",
        num_cores=2, num_subcores=16)


def _dense_gate_body(x_ref, w1_ref, w2_ref, wg_ref, b1_ref, b2_ref,
                     h2_ref, pinfo_ref):
    f32 = jnp.float32
    h1 = jnp.dot(x_ref[...], w1_ref[...], preferred_element_type=f32)
    h1 = jnp.maximum(h1 + b1_ref[...], 0.0).astype(jnp.bfloat16)
    h2 = jnp.dot(h1, w2_ref[...], preferred_element_type=f32)
    h2 = jnp.maximum(h2 + b2_ref[...], 0.0)
    h2_ref[...] = h2.astype(jnp.bfloat16)
    gl = jnp.dot(h2, wg_ref[...], preferred_element_type=f32)
    lane = jax.lax.broadcasted_iota(jnp.int32, gl.shape, 1)
    gl = jnp.where(lane < E, gl, NEG)
    m1 = jnp.max(gl, axis=-1, keepdims=True)
    i1 = jnp.min(jnp.where(gl == m1, lane, 127), axis=-1, keepdims=True)
    gl2 = jnp.where(lane == i1, NEG, gl)
    m2 = jnp.max(gl2, axis=-1, keepdims=True)
    i2 = jnp.min(jnp.where(gl2 == m2, lane, 127), axis=-1, keepdims=True)
    s = jnp.sum(jnp.exp(gl - m1), axis=-1, keepdims=True)
    v1 = 1.0 / s
    v2 = jnp.exp(m2 - m1) / s
    denom = v1 + v2 + 1e-9
    w1w = v1 / denom
    w2w = v2 / denom
    li = lane[: x_ref.shape[0], :]
    pinfo = jnp.where(li == 0, i1.astype(f32),
            jnp.where(li == 1, i2.astype(f32),
            jnp.where(li == 2, w1w,
            jnp.where(li == 3, w2w, 0.0))))
    pinfo_ref[...] = pinfo


def _sort_body(pinfo_ref, dest_ref, pocnt_ref, excl_s, tot_s, po_s):
    f32 = jnp.float32
    ph = pl.program_id(0)
    j = pl.program_id(1)
    blk = pinfo_ref[...]
    ec = jnp.where(j < 8, blk[:, 0:1], blk[:, 1:2]).astype(jnp.int32)
    lanei = jax.lax.broadcasted_iota(jnp.int32, (CH, 128), 1)
    oh32 = (ec == lanei).astype(f32)

    @pl.when((ph == 0) & (j == 0))
    def _():
        tot_s[...] = jnp.zeros_like(tot_s)

    @pl.when(ph == 0)
    def _():
        excl_s[pl.ds(j, 1), :] = tot_s[...]
        tot_s[...] = tot_s[...] + jnp.sum(oh32, axis=0, keepdims=True)

    @pl.when((ph == 1) & (j == 0))
    def _():
        tot = tot_s[...]
        padc = jnp.floor((tot + 511.0) * (1.0 / 512.0)) * 512.0
        li = jax.lax.broadcasted_iota(jnp.int32, (128, 128), 0)
        lj = jax.lax.broadcasted_iota(jnp.int32, (128, 128), 1)
        t128 = (li < lj).astype(f32)
        po_s[...] = jnp.dot(padc, t128, preferred_element_type=f32)
        pocnt_ref[0:1, :] = po_s[...].astype(jnp.int32)
        pocnt_ref[1:2, :] = tot.astype(jnp.int32)

    @pl.when(ph == 1)
    def _():
        ti = jax.lax.broadcasted_iota(jnp.int32, (CH, CH), 0)
        tj = jax.lax.broadcasted_iota(jnp.int32, (CH, CH), 1)
        tril = (tj < ti).astype(jnp.bfloat16)
        rank = jnp.dot(tril, oh32.astype(jnp.bfloat16),
                       preferred_element_type=f32)
        base = po_s[...] + excl_s[pl.ds(j, 1), :]
        destf = jnp.sum(oh32 * (rank + base), axis=1, keepdims=True)
        dest_ref[...] = destf.astype(jnp.int32).reshape(1, 4, 128)


@functools.cache
def _make_sc_dispatch():
    @functools.partial(
        pl.kernel,
        out_type=jax.ShapeDtypeStruct((HS_PAD, D2), jnp.int32),
        mesh=_mesh(),
        scratch_types=[
            pltpu.VMEM((4, 64), jnp.int32),
            pltpu.VMEM((64, D2), jnp.int32),
            pltpu.SemaphoreType.DMA,
        ],
    )
    def _sc_dispatch(h2, dest3, hs, idx_v, buf, sem):
        wid = lax.axis_index("s") * 2 + lax.axis_index("c")
        pltpu.sync_copy(dest3.at[wid], idx_v)
        row0 = (wid * PW) % N_TOK
        for sub in range(4):
            pltpu.sync_copy(h2.at[pl.ds(row0 + sub * 64, 64)], buf)
            pltpu.async_copy(buf, hs.at[idx_v.at[sub]], sem).wait()

    return _sc_dispatch


def _expert_body(scal_ref, hs_ref, we_ref, be_ref, y_ref):
    lo = pl.program_id(0) * BE
    xb = hs_ref[...]
    for e in range(E):
        @pl.when((scal_ref[e] <= lo) & (lo < scal_ref[e] + scal_ref[8 + e]))
        def _():
            y_ref[...] = (jnp.dot(xb, we_ref[e],
                                  preferred_element_type=jnp.float32)
                          + be_ref[e][None, :]).astype(jnp.bfloat16)


@functools.cache
def _make_sc_combine():
    @functools.partial(
        pl.kernel,
        out_type=(jax.ShapeDtypeStruct((N_TOK, D2), jnp.int32),
                  jax.ShapeDtypeStruct((N_TOK, D2), jnp.int32)),
        mesh=_mesh(),
        scratch_types=[
            pltpu.VMEM((2, 64), jnp.int32),
            pltpu.VMEM((64, D2), jnp.int32),
            pltpu.SemaphoreType.DMA,
        ],
    )
    def _sc_combine(y, s13, s23, ys1, ys2, idx_v, buf, sem):
        wid = lax.axis_index("s") * 2 + lax.axis_index("c")
        tb = wid * TW
        for sref, oref in ((s13, ys1), (s23, ys2)):
            pltpu.sync_copy(sref.at[wid], idx_v)
            for sub in range(2):
                pltpu.async_copy(y.at[idx_v.at[sub]], buf, sem).wait()
                pltpu.sync_copy(buf, oref.at[pl.ds(tb + sub * 64, 64)])

    return _sc_combine


def _final_body(ys1_ref, ys2_ref, pinfo_ref, wp_ref, bp_ref, out_ref):
    f32 = jnp.float32
    w1w = pinfo_ref[:, 2:3]
    w2w = pinfo_ref[:, 3:4]
    hm = (w1w * ys1_ref[...].astype(f32)
          + w2w * ys2_ref[...].astype(f32)).astype(jnp.bfloat16)
    logits = jnp.dot(hm, wp_ref[...], preferred_element_type=f32) + bp_ref[...]
    m = jnp.max(logits, axis=-1, keepdims=True)
    lse = jnp.log(jnp.sum(jnp.exp(logits - m), axis=-1, keepdims=True))
    out_ref[...] = logits - m - lse


@jax.jit
def kernel(x, W1, b1, W2, b2, Wg, We, be, Wp, bp):
    f32 = jnp.float32
    bf16 = jnp.bfloat16
    i32 = jnp.int32
    xf = x.reshape(N_TOK, D).astype(bf16)
    wg_pad = jnp.zeros((D, 128), f32).at[:, :E].set(Wg)
    wp_pad = jnp.zeros((D, 1024), bf16).at[:, :NC].set(Wp.astype(bf16))
    bp_pad = jnp.full((1, 1024), NEG, f32).at[0, :NC].set(bp)
    full = lambda s: pl.BlockSpec(s, lambda i: tuple(0 for _ in s))

    h2, pinfo = pl.pallas_call(
        _dense_gate_body,
        grid=(N_TOK // BT,),
        in_specs=[
            pl.BlockSpec((BT, D), lambda i: (i, 0)),
            full((D, D)), full((D, D)), full((D, 128)),
            full((1, D)), full((1, D)),
        ],
        out_specs=[pl.BlockSpec((BT, D), lambda i: (i, 0)),
                   pl.BlockSpec((BT, 128), lambda i: (i, 0))],
        out_shape=[jax.ShapeDtypeStruct((N_TOK, D), bf16),
                   jax.ShapeDtypeStruct((N_TOK, 128), f32)],
        compiler_params=pltpu.CompilerParams(
            dimension_semantics=("arbitrary",)),
    )(xf, W1.astype(bf16), W2.astype(bf16), wg_pad,
      b1.reshape(1, D), b2.reshape(1, D))

    dest, pocnt = pl.pallas_call(
        _sort_body,
        grid=(2, N_TOK // CH * 2),
        in_specs=[pl.BlockSpec((CH, 128), lambda i, j: (lax.rem(j, 8), 0))],
        out_specs=[pl.BlockSpec((1, 4, 128), lambda i, j: (i * j, 0, 0)),
                   pl.BlockSpec((8, 128), lambda i, j: (0, 0))],
        out_shape=[jax.ShapeDtypeStruct((16, 4, 128), i32),
                   jax.ShapeDtypeStruct((8, 128), i32)],
        scratch_shapes=[pltpu.VMEM((16, 128), f32),
                        pltpu.VMEM((1, 128), f32),
                        pltpu.VMEM((1, 128), f32)],
        compiler_params=pltpu.CompilerParams(
            dimension_semantics=("arbitrary", "arbitrary")),
    )(pinfo)

    dcol = dest.reshape(NPAIR)
    dest3 = dcol.reshape(NW, 4, 64)
    s13 = dcol[:N_TOK].reshape(NW, 2, 64)
    s23 = dcol[N_TOK:].reshape(NW, 2, 64)
    scal = jnp.concatenate([pocnt[0, :6], jnp.zeros((2,), i32),
                            pocnt[1, :6], jnp.zeros((2,), i32)])

    h2i = lax.bitcast_convert_type(h2.reshape(N_TOK, D2, 2), i32)
    hsi = _make_sc_dispatch()(h2i, dest3)
    hs = lax.bitcast_convert_type(hsi, bf16).reshape(HS_PAD, D)

    y = pl.pallas_call(
        _expert_body,
        grid_spec=pltpu.PrefetchScalarGridSpec(
            num_scalar_prefetch=1,
            grid=(HS_PAD // BE,),
            in_specs=[
                pl.BlockSpec((BE, D), lambda g, s: (g, 0)),
                pl.BlockSpec((E, D, D), lambda g, s: (0, 0, 0)),
                pl.BlockSpec((E, D), lambda g, s: (0, 0)),
            ],
            out_specs=pl.BlockSpec((BE, D), lambda g, s: (g, 0)),
        ),
        out_shape=jax.ShapeDtypeStruct((HS_PAD, D), bf16),
        compiler_params=pltpu.CompilerParams(
            dimension_semantics=("arbitrary",)),
    )(scal, hs, We.astype(bf16), be)

    yi = lax.bitcast_convert_type(y.reshape(HS_PAD, D2, 2), i32)
    ys1i, ys2i = _make_sc_combine()(yi, s13, s23)
    ys1 = lax.bitcast_convert_type(ys1i, bf16).reshape(N_TOK, D)
    ys2 = lax.bitcast_convert_type(ys2i, bf16).reshape(N_TOK, D)

    out = pl.pallas_call(
        _final_body,
        grid=(N_TOK // BT,),
        in_specs=[
            pl.BlockSpec((BT, D), lambda i: (i, 0)),
            pl.BlockSpec((BT, D), lambda i: (i, 0)),
            pl.BlockSpec((BT, 128), lambda i: (i, 0)),
            full((D, 1024)), full((1, 1024)),
        ],
        out_specs=pl.BlockSpec((BT, 1024), lambda i: (i, 0)),
        out_shape=jax.ShapeDtypeStruct((N_TOK, 1024), f32),
        compiler_params=pltpu.CompilerParams(
            dimension_semantics=("arbitrary",)),
    )(ys1, ys2, pinfo, wp_pad, bp_pad)
    return out[:, :NC]


# speedup vs baseline: 3.3961x; 3.3961x over previous
import functools

import jax
import jax.numpy as jnp
from jax import lax
from jax.experimental import pallas as pl
from jax.experimental.pallas import tpu as pltpu
from jax.experimental.pallas import tpu_sc as plsc

N_TOK = 4096
NH = 2048
D = 1024
E = 6
NC = 1000
BT = 512
CH = 512
NPH = 2 * NH
NCHUNK = NPH // CH
BE = 512
HS_H = NPH + E * BE
NW = 32
PW = NPH // NW
TW = NH // NW
NEG = -1e30


def _mesh():
    return plsc.VectorSubcoreMesh(
        core_axis_name="c", subcore_axis_name="s",
        num_cores=2, num_subcores=16)


def _dense_gate_body(x_ref, w1_ref, w2_ref, wg_ref, b1_ref, b2_ref,
                     h2_ref, pinfo_ref):
    f32 = jnp.float32
    h1 = jnp.dot(x_ref[...], w1_ref[...], preferred_element_type=f32)
    h1 = jnp.maximum(h1 + b1_ref[...], 0.0).astype(jnp.bfloat16)
    h2 = jnp.dot(h1, w2_ref[...], preferred_element_type=f32)
    h2 = jnp.maximum(h2 + b2_ref[...], 0.0)
    h2_ref[...] = h2
    gl = jnp.dot(h2, wg_ref[...], preferred_element_type=f32)
    lane = jax.lax.broadcasted_iota(jnp.int32, gl.shape, 1)
    gl = jnp.where(lane < E, gl, NEG)
    m1 = jnp.max(gl, axis=-1, keepdims=True)
    i1 = jnp.min(jnp.where(gl == m1, lane, 127), axis=-1, keepdims=True)
    gl2 = jnp.where(lane == i1, NEG, gl)
    m2 = jnp.max(gl2, axis=-1, keepdims=True)
    i2 = jnp.min(jnp.where(gl2 == m2, lane, 127), axis=-1, keepdims=True)
    s = jnp.sum(jnp.exp(gl - m1), axis=-1, keepdims=True)
    v1 = 1.0 / s
    v2 = jnp.exp(m2 - m1) / s
    denom = v1 + v2 + 1e-9
    w1w = v1 / denom
    w2w = v2 / denom
    li = lane[: x_ref.shape[0], :]
    pinfo = jnp.where(li == 0, i1.astype(f32),
            jnp.where(li == 1, i2.astype(f32),
            jnp.where(li == 2, w1w,
            jnp.where(li == 3, w2w, 0.0))))
    pinfo_ref[...] = pinfo


def _sort_body(pinfo_ref, dest_ref, pocnt_ref, excl_s, tot_s, po_s):
    f32 = jnp.float32
    ph = pl.program_id(0)
    j = pl.program_id(1)
    blk = pinfo_ref[...]
    ec = jnp.where(j < NCHUNK // 2, blk[:, 0:1], blk[:, 1:2]).astype(jnp.int32)
    lanei = jax.lax.broadcasted_iota(jnp.int32, (CH, 128), 1)
    oh32 = (ec == lanei).astype(f32)

    @pl.when((ph == 0) & (j == 0))
    def _():
        tot_s[...] = jnp.zeros_like(tot_s)

    @pl.when(ph == 0)
    def _():
        excl_s[pl.ds(j, 1), :] = tot_s[...]
        tot_s[...] = tot_s[...] + jnp.sum(oh32, axis=0, keepdims=True)

    @pl.when((ph == 1) & (j == 0))
    def _():
        tot = tot_s[...]
        padc = jnp.floor((tot + (BE - 1.0)) * (1.0 / BE)) * BE
        li = jax.lax.broadcasted_iota(jnp.int32, (128, 128), 0)
        lj = jax.lax.broadcasted_iota(jnp.int32, (128, 128), 1)
        t128 = (li < lj).astype(f32)
        po_s[...] = jnp.dot(padc, t128, preferred_element_type=f32)
        pocnt_ref[0:1, :] = po_s[...].astype(jnp.int32)
        pocnt_ref[1:2, :] = tot.astype(jnp.int32)

    @pl.when(ph == 1)
    def _():
        ti = jax.lax.broadcasted_iota(jnp.int32, (CH, CH), 0)
        tj = jax.lax.broadcasted_iota(jnp.int32, (CH, CH), 1)
        tril = (tj < ti).astype(jnp.bfloat16)
        rank = jnp.dot(tril, oh32.astype(jnp.bfloat16),
                       preferred_element_type=f32)
        base = po_s[...] + excl_s[pl.ds(j, 1), :]
        destf = jnp.sum(oh32 * (rank + base), axis=1, keepdims=True)
        dest_ref[...] = destf.astype(jnp.int32).reshape(1, 4, 128)


@functools.cache
def _make_sc_dispatch():
    @functools.partial(
        pl.kernel,
        out_type=jax.ShapeDtypeStruct((HS_H, D), jnp.float32),
        mesh=_mesh(),
        scratch_types=[
            pltpu.VMEM((4, 32), jnp.int32),
            pltpu.VMEM((32, D), jnp.float32),
            pltpu.VMEM((32, D), jnp.float32),
            pltpu.SemaphoreType.DMA,
            pltpu.SemaphoreType.DMA,
        ],
    )
    def _sc_dispatch(h2, dest4, hs, idx_v, buf0, buf1, rsem, ssem):
        wid = lax.axis_index("s") * 2 + lax.axis_index("c")
        pltpu.sync_copy(dest4.at[wid], idx_v)
        row0 = (wid * PW) % NH
        bufs = (buf0, buf1)
        reads = [None] * 4
        scats = [None] * 4
        for s in range(2):
            reads[s] = pltpu.async_copy(
                h2.at[pl.ds(row0 + s * 32, 32)], bufs[s], rsem)
        for s in range(4):
            reads[s].wait()
            scats[s] = pltpu.async_copy(bufs[s % 2], hs.at[idx_v.at[s]], ssem)
            if s + 2 < 4:
                scats[s].wait()
                reads[s + 2] = pltpu.async_copy(
                    h2.at[pl.ds(row0 + (s + 2) * 32, 32)], bufs[s % 2], rsem)
        scats[2].wait()
        scats[3].wait()

    return _sc_dispatch


def _expert_body(scal_ref, hs_ref, we_ref, be_ref, y_ref):
    lo = pl.program_id(0) * BE
    xb = hs_ref[...].astype(jnp.bfloat16)
    for e in range(E):
        @pl.when((scal_ref[e] <= lo) & (lo < scal_ref[e] + scal_ref[8 + e]))
        def _():
            y_ref[...] = (jnp.dot(xb, we_ref[e],
                                  preferred_element_type=jnp.float32)
                          + be_ref[e][None, :])


@functools.cache
def _make_sc_combine():
    @functools.partial(
        pl.kernel,
        out_type=(jax.ShapeDtypeStruct((NH, D), jnp.float32),
                  jax.ShapeDtypeStruct((NH, D), jnp.float32)),
        mesh=_mesh(),
        scratch_types=[
            pltpu.VMEM((2, 32), jnp.int32),
            pltpu.VMEM((32, D), jnp.float32),
            pltpu.VMEM((32, D), jnp.float32),
            pltpu.SemaphoreType.DMA,
            pltpu.SemaphoreType.DMA,
        ],
    )
    def _sc_combine(y, s14, s24, ys1, ys2, idx_v, buf0, buf1, gsem, wsem):
        wid = lax.axis_index("s") * 2 + lax.axis_index("c")
        tb = wid * TW
        bufs = (buf0, buf1)
        for sref, oref in ((s14, ys1), (s24, ys2)):
            pltpu.sync_copy(sref.at[wid], idx_v)
            g0 = pltpu.async_copy(y.at[idx_v.at[0]], bufs[0], gsem)
            g1 = pltpu.async_copy(y.at[idx_v.at[1]], bufs[1], gsem)
            g0.wait()
            w0 = pltpu.async_copy(bufs[0], oref.at[pl.ds(tb, 32)], wsem)
            g1.wait()
            w1 = pltpu.async_copy(bufs[1], oref.at[pl.ds(tb + 32, 32)], wsem)
            w0.wait()
            w1.wait()

    return _sc_combine


def _final_body(ys1_ref, ys2_ref, pinfo_ref, wp_ref, bp_ref, out_ref):
    f32 = jnp.float32
    w1w = pinfo_ref[:, 2:3]
    w2w = pinfo_ref[:, 3:4]
    hm = (w1w * ys1_ref[...] + w2w * ys2_ref[...]).astype(jnp.bfloat16)
    logits = jnp.dot(hm, wp_ref[...], preferred_element_type=f32) + bp_ref[...]
    m = jnp.max(logits, axis=-1, keepdims=True)
    lse = jnp.log(jnp.sum(jnp.exp(logits - m), axis=-1, keepdims=True))
    out_ref[...] = logits - m - lse


def _half(xh, w1b, w2b, wgp, b1r, b2r, web, be, wpp, bpp):
    f32 = jnp.float32
    bf16 = jnp.bfloat16
    i32 = jnp.int32
    full = lambda s: pl.BlockSpec(s, lambda i: tuple(0 for _ in s))

    h2, pinfo = pl.pallas_call(
        _dense_gate_body,
        grid=(NH // BT,),
        in_specs=[
            pl.BlockSpec((BT, D), lambda i: (i, 0)),
            full((D, D)), full((D, D)), full((D, 128)),
            full((1, D)), full((1, D)),
        ],
        out_specs=[pl.BlockSpec((BT, D), lambda i: (i, 0)),
                   pl.BlockSpec((BT, 128), lambda i: (i, 0))],
        out_shape=[jax.ShapeDtypeStruct((NH, D), f32),
                   jax.ShapeDtypeStruct((NH, 128), f32)],
        compiler_params=pltpu.CompilerParams(
            dimension_semantics=("arbitrary",)),
    )(xh, w1b, w2b, wgp, b1r, b2r)

    dest, pocnt = pl.pallas_call(
        _sort_body,
        grid=(2, NCHUNK),
        in_specs=[pl.BlockSpec(
            (CH, 128), lambda i, j: (lax.rem(j, NCHUNK // 2), 0))],
        out_specs=[pl.BlockSpec((1, 4, 128), lambda i, j: (i * j, 0, 0)),
                   pl.BlockSpec((8, 128), lambda i, j: (0, 0))],
        out_shape=[jax.ShapeDtypeStruct((NCHUNK, 4, 128), i32),
                   jax.ShapeDtypeStruct((8, 128), i32)],
        scratch_shapes=[pltpu.VMEM((NCHUNK, 128), f32),
                        pltpu.VMEM((1, 128), f32),
                        pltpu.VMEM((1, 128), f32)],
        compiler_params=pltpu.CompilerParams(
            dimension_semantics=("arbitrary", "arbitrary")),
    )(pinfo)

    dcol = dest.reshape(NPH)
    dest4 = dcol.reshape(NW, 4, 32)
    s14 = dcol[:NH].reshape(NW, 2, 32)
    s24 = dcol[NH:].reshape(NW, 2, 32)
    scal = jnp.concatenate([pocnt[0, :6], jnp.zeros((2,), i32),
                            pocnt[1, :6], jnp.zeros((2,), i32)])

    hs = _make_sc_dispatch()(h2, dest4)

    y = pl.pallas_call(
        _expert_body,
        grid_spec=pltpu.PrefetchScalarGridSpec(
            num_scalar_prefetch=1,
            grid=(HS_H // BE,),
            in_specs=[
                pl.BlockSpec((BE, D), lambda g, s: (g, 0)),
                pl.BlockSpec((E, D, D), lambda g, s: (0, 0, 0)),
                pl.BlockSpec((E, D), lambda g, s: (0, 0)),
            ],
            out_specs=pl.BlockSpec((BE, D), lambda g, s: (g, 0)),
        ),
        out_shape=jax.ShapeDtypeStruct((HS_H, D), f32),
        compiler_params=pltpu.CompilerParams(
            dimension_semantics=("arbitrary",)),
    )(scal, hs, web, be)

    ys1, ys2 = _make_sc_combine()(y, s14, s24)

    out = pl.pallas_call(
        _final_body,
        grid=(NH // BT,),
        in_specs=[
            pl.BlockSpec((BT, D), lambda i: (i, 0)),
            pl.BlockSpec((BT, D), lambda i: (i, 0)),
            pl.BlockSpec((BT, 128), lambda i: (i, 0)),
            full((D, 1024)), full((1, 1024)),
        ],
        out_specs=pl.BlockSpec((BT, 1024), lambda i: (i, 0)),
        out_shape=jax.ShapeDtypeStruct((NH, 1024), f32),
        compiler_params=pltpu.CompilerParams(
            dimension_semantics=("arbitrary",)),
    )(ys1, ys2, pinfo, wpp, bpp)
    return out


@jax.jit
def kernel(x, W1, b1, W2, b2, Wg, We, be, Wp, bp):
    f32 = jnp.float32
    bf16 = jnp.bfloat16
    xf = x.reshape(N_TOK, D).astype(bf16)
    wg_pad = jnp.zeros((D, 128), f32).at[:, :E].set(Wg)
    wp_pad = jnp.zeros((D, 1024), bf16).at[:, :NC].set(Wp.astype(bf16))
    bp_pad = jnp.full((1, 1024), NEG, f32).at[0, :NC].set(bp)
    w1b = W1.astype(bf16)
    w2b = W2.astype(bf16)
    web = We.astype(bf16)
    b1r = b1.reshape(1, D)
    b2r = b2.reshape(1, D)

    outs = [
        _half(xf[h * NH:(h + 1) * NH], w1b, w2b, wg_pad, b1r, b2r,
              web, be, wp_pad, bp_pad)
        for h in range(2)
    ]
    out = jnp.concatenate(outs, axis=0)
    return out[:, :NC]


# speedup vs baseline: 5.9456x; 1.7507x over previous
import functools

import jax
import jax.numpy as jnp
from jax.experimental import pallas as pl
from jax.experimental.pallas import tpu as pltpu

N_TOK = 4096
D = 1024
E = 6
NC = 1000
BT = 512
NEG = -1e30


def _top2(gl):
    lane = jax.lax.broadcasted_iota(jnp.int32, gl.shape, 1)
    m1 = jnp.max(gl, axis=-1, keepdims=True)
    i1 = jnp.min(jnp.where(gl == m1, lane, 127), axis=-1, keepdims=True)
    gl2 = jnp.where(lane == i1, NEG, gl)
    m2 = jnp.max(gl2, axis=-1, keepdims=True)
    i2 = jnp.min(jnp.where(gl2 == m2, lane, 127), axis=-1, keepdims=True)
    s = jnp.sum(jnp.exp(gl - m1), axis=-1, keepdims=True)
    v1 = 1.0 / s
    v2 = jnp.exp(m2 - m1) / s
    denom = v1 + v2 + 1e-9
    return i1, i2, v1 / denom, v2 / denom


def _fused_body(x_ref, w1_ref, w2_ref, wg_ref, we_ref, be_ref, b1_ref,
                b2_ref, wp_ref, bp_ref, out_ref):
    f32 = jnp.float32
    xb = x_ref[...]
    h1 = jnp.dot(xb, w1_ref[...], preferred_element_type=f32) + b1_ref[...]
    h1 = jnp.maximum(h1, 0.0).astype(jnp.bfloat16)
    h2 = jnp.dot(h1, w2_ref[...], preferred_element_type=f32) + b2_ref[...]
    h2 = jnp.maximum(h2, 0.0)
    h2b = h2.astype(jnp.bfloat16)
    gl = jnp.dot(h2, wg_ref[...], preferred_element_type=f32)
    lane = jax.lax.broadcasted_iota(jnp.int32, gl.shape, 1)
    gl = jnp.where(lane < E, gl, NEG)
    i1, i2, w1w, w2w = _top2(gl)
    acc = jnp.zeros((xb.shape[0], D), f32)
    for e in range(E):
        we = (jnp.where(i1 == e, w1w, 0.0) + jnp.where(i2 == e, w2w, 0.0))
        eo = jnp.dot(h2b, we_ref[e], preferred_element_type=f32) + be_ref[e][None, :]
        acc = acc + we * eo
    logits = jnp.dot(acc.astype(jnp.bfloat16), wp_ref[...],
                     preferred_element_type=f32) + bp_ref[...]
    m = jnp.max(logits, axis=-1, keepdims=True)
    lse = jnp.log(jnp.sum(jnp.exp(logits - m), axis=-1, keepdims=True))
    out_ref[...] = logits - m - lse


@jax.jit
def kernel(x, W1, b1, W2, b2, Wg, We, be, Wp, bp):
    bf16 = jnp.bfloat16
    xf = x.reshape(N_TOK, D).astype(bf16)
    wg_pad = jnp.zeros((D, 128), jnp.float32).at[:, :E].set(Wg)
    wp_pad = Wp.astype(bf16)
    bp_pad = bp.reshape(1, NC)

    full = lambda s: pl.BlockSpec(s, lambda i: tuple(0 for _ in s))
    out = pl.pallas_call(
        _fused_body,
        grid=(N_TOK // BT,),
        in_specs=[
            pl.BlockSpec((BT, D), lambda i: (i, 0)),
            full((D, D)), full((D, D)), full((D, 128)),
            full((E, D, D)), full((E, D)),
            full((1, D)), full((1, D)),
            full((D, NC)), full((1, NC)),
        ],
        out_specs=pl.BlockSpec((BT, NC), lambda i: (i, 0)),
        out_shape=jax.ShapeDtypeStruct((N_TOK, NC), jnp.float32),
        compiler_params=pltpu.CompilerParams(
            dimension_semantics=("arbitrary",)),
    )(xf, W1.astype(bf16), W2.astype(bf16), wg_pad, We.astype(bf16),
      be, b1.reshape(1, D), b2.reshape(1, D), wp_pad, bp_pad)
    return out


# speedup vs baseline: 5.9461x; 1.0001x over previous
import functools

import jax
import jax.numpy as jnp
from jax.experimental import pallas as pl
from jax.experimental.pallas import tpu as pltpu

N_TOK = 4096
D = 1024
E = 6
NC = 1000
BT = 1024
NEG = -1e30


def _top2(gl):
    lane = jax.lax.broadcasted_iota(jnp.int32, gl.shape, 1)
    m1 = jnp.max(gl, axis=-1, keepdims=True)
    i1 = jnp.min(jnp.where(gl == m1, lane, 127), axis=-1, keepdims=True)
    gl2 = jnp.where(lane == i1, NEG, gl)
    m2 = jnp.max(gl2, axis=-1, keepdims=True)
    i2 = jnp.min(jnp.where(gl2 == m2, lane, 127), axis=-1, keepdims=True)
    s = jnp.sum(jnp.exp(gl - m1), axis=-1, keepdims=True)
    v1 = 1.0 / s
    v2 = jnp.exp(m2 - m1) / s
    denom = v1 + v2 + 1e-9
    return i1, i2, v1 / denom, v2 / denom


def _fused_body(x_ref, w1_ref, w2_ref, wg_ref, we_ref, be_ref, b1_ref,
                b2_ref, wp_ref, bp_ref, out_ref):
    f32 = jnp.float32
    xb = x_ref[...]
    h1 = jnp.dot(xb, w1_ref[...], preferred_element_type=f32) + b1_ref[...]
    h1 = jnp.maximum(h1, 0.0).astype(jnp.bfloat16)
    h2 = jnp.dot(h1, w2_ref[...], preferred_element_type=f32) + b2_ref[...]
    h2 = jnp.maximum(h2, 0.0)
    h2b = h2.astype(jnp.bfloat16)
    gl = jnp.dot(h2, wg_ref[...], preferred_element_type=f32)
    lane = jax.lax.broadcasted_iota(jnp.int32, gl.shape, 1)
    gl = jnp.where(lane < E, gl, NEG)
    i1, i2, w1w, w2w = _top2(gl)
    acc = jnp.zeros((xb.shape[0], D), f32)
    for e in range(E):
        we = (jnp.where(i1 == e, w1w, 0.0) + jnp.where(i2 == e, w2w, 0.0))
        eo = jnp.dot(h2b, we_ref[e], preferred_element_type=f32) + be_ref[e][None, :]
        acc = acc + we * eo
    logits = jnp.dot(acc.astype(jnp.bfloat16), wp_ref[...],
                     preferred_element_type=f32) + bp_ref[...]
    m = jnp.max(logits, axis=-1, keepdims=True)
    lse = jnp.log(jnp.sum(jnp.exp(logits - m), axis=-1, keepdims=True))
    out_ref[...] = logits - m - lse


@jax.jit
def kernel(x, W1, b1, W2, b2, Wg, We, be, Wp, bp):
    bf16 = jnp.bfloat16
    xf = x.reshape(N_TOK, D).astype(bf16)
    wg_pad = jnp.zeros((D, 128), jnp.float32).at[:, :E].set(Wg)
    wp_pad = Wp.astype(bf16)
    bp_pad = bp.reshape(1, NC)

    full = lambda s: pl.BlockSpec(s, lambda i: tuple(0 for _ in s))
    out = pl.pallas_call(
        _fused_body,
        grid=(N_TOK // BT,),
        in_specs=[
            pl.BlockSpec((BT, D), lambda i: (i, 0)),
            full((D, D)), full((D, D)), full((D, 128)),
            full((E, D, D)), full((E, D)),
            full((1, D)), full((1, D)),
            full((D, NC)), full((1, NC)),
        ],
        out_specs=pl.BlockSpec((BT, NC), lambda i: (i, 0)),
        out_shape=jax.ShapeDtypeStruct((N_TOK, NC), jnp.float32),
        compiler_params=pltpu.CompilerParams(
            dimension_semantics=("arbitrary",)),
    )(xf, W1.astype(bf16), W2.astype(bf16), wg_pad, We.astype(bf16),
      be, b1.reshape(1, D), b2.reshape(1, D), wp_pad, bp_pad)
    return out
